# baseline (speedup 1.0000x reference)
import functools

import jax
import jax.numpy as jnp
from jax import lax
from jax.experimental import pallas as pl
from jax.experimental.pallas import tpu as pltpu
from jax.experimental.pallas import tpu_sc as plsc

LANES = 16
CHUNK = 16
NBUF = 2


def _rsqrt_vec(v):
    i = lax.bitcast_convert_type(v, jnp.int32)
    y = lax.bitcast_convert_type(jnp.int32(0x5F3759DF) - (i >> 1), jnp.float32)
    for _ in range(2):
        y = y * (1.5 - 0.5 * v * y * y)
    return y


@functools.lru_cache(maxsize=None)
def _build_sc_kernel(n_rows, seq, hidden):
    info = plsc.get_sparse_core_info()
    nc, ns = info.num_cores, info.num_subcores
    nw = nc * ns
    assert n_rows % (nw * CHUNK) == 0
    rows_per_w = n_rows // nw
    n_chunks = rows_per_w // CHUNK
    assert n_chunks % NBUF == 0 and n_chunks >= 2 * NBUF
    nv = hidden // LANES

    mesh = plsc.VectorSubcoreMesh(core_axis_name="c", subcore_axis_name="s")

    @functools.partial(
        pl.kernel,
        mesh=mesh,
        compiler_params=pltpu.CompilerParams(needs_layout_passes=False),
        out_type=jax.ShapeDtypeStruct((n_rows, hidden), jnp.float32),
        scratch_types=[
            pltpu.VMEM((rows_per_w,), jnp.int32),
            [pltpu.VMEM((CHUNK, hidden), jnp.float32)] * NBUF,
            [pltpu.VMEM((CHUNK, hidden), jnp.float32)] * NBUF,
            [pltpu.VMEM((CHUNK, hidden), jnp.float32)] * NBUF,
            pltpu.VMEM((hidden,), jnp.float32),
            pltpu.VMEM((hidden,), jnp.float32),
            [pltpu.SemaphoreType.DMA] * NBUF,
            [pltpu.SemaphoreType.DMA] * NBUF,
            [pltpu.SemaphoreType.DMA] * NBUF,
        ],
    )
    def body(ids_hbm, tok_hbm, pos_hbm, gamma_hbm, beta_hbm, out_hbm,
             idx_all, xs, ps, os_, g_v, b_v, gsems, psems, osems):
        wid = lax.axis_index("s") * nc + lax.axis_index("c")
        row0 = wid * rows_per_w
        pos0 = lax.rem(row0, seq)

        pltpu.sync_copy(ids_hbm.at[pl.ds(row0, rows_per_w)], idx_all)
        pltpu.sync_copy(gamma_hbm, g_v)
        pltpu.sync_copy(beta_hbm, b_v)

        zero = jnp.zeros((LANES,), jnp.float32)
        inv_h = jnp.float32(1.0 / hidden)

        def start_pos(ci, b):
            pltpu.async_copy(
                pos_hbm.at[pl.ds(pos0 + ci * CHUNK, CHUNK)], ps[b], psems[b])

        def wait_pos(b):
            pltpu.make_async_copy(pos_hbm.at[pl.ds(0, CHUNK)], ps[b],
                                  psems[b]).wait()

        def start_ga(ci, b):
            pltpu.async_copy(
                tok_hbm.at[idx_all.at[pl.ds(ci * CHUNK, CHUNK)]], xs[b],
                gsems[b])

        def wait_ga(b):
            pltpu.make_async_copy(
                tok_hbm.at[idx_all.at[pl.ds(0, CHUNK)]], xs[b],
                gsems[b]).wait()

        def start_out(ci, b):
            pltpu.async_copy(os_[b], out_hbm.at[pl.ds(row0 + ci * CHUNK, CHUNK)],
                             osems[b])

        def wait_out(b):
            pltpu.make_async_copy(os_[b], out_hbm.at[pl.ds(0, CHUNK)],
                                  osems[b]).wait()

        for b in range(NBUF):
            start_pos(b, b)
        start_ga(0, 0)

        dev = [zero] * 2
        for i in range(nv):
            sl = pl.ds(i * LANES, LANES)
            dev[i % 2] = dev[i % 2] + (jnp.abs(g_v[sl] - 1.0) + jnp.abs(b_v[sl]))
        is_identity = jnp.sum(dev[0] + dev[1]) == 0.0

        def compute_chunk(b, fast):
            @plsc.parallel_loop(0, CHUNK, unroll=3)
            def row_body(r):
                s = [zero] * 2
                ss = [zero] * 2
                for i in range(nv):
                    sl = pl.ds(i * LANES, LANES)
                    t = xs[b][r, sl] + ps[b][r, sl]
                    os_[b][r, sl] = t
                    s[i % 2] = s[i % 2] + t
                    ss[i % 2] = ss[i % 2] + t * t
                stot = s[0] + s[1]
                sstot = ss[0] + ss[1]
                mu = jnp.sum(stot) * inv_h
                var = jnp.sum(sstot) * inv_h - mu * mu
                rstd = _rsqrt_vec(zero + (var + jnp.float32(1e-12)))
                c = mu * rstd
                for i in range(nv):
                    sl = pl.ds(i * LANES, LANES)
                    y = os_[b][r, sl] * rstd - c
                    if not fast:
                        y = y * g_v[sl] + b_v[sl]
                    os_[b][r, sl] = y

        def outer(g, _):
            for b in range(NBUF):
                ci = g * NBUF + b
                nb = (b + 1) % NBUF

                @pl.when(ci + 1 < n_chunks)
                def _():
                    start_ga(ci + 1, nb)

                wait_ga(b)
                wait_pos(b)

                @pl.when(ci >= NBUF)
                def _():
                    wait_out(b)

                @pl.when(is_identity)
                def _():
                    compute_chunk(b, True)

                @pl.when(jnp.logical_not(is_identity))
                def _():
                    compute_chunk(b, False)

                start_out(ci, b)

                @pl.when(ci + NBUF < n_chunks)
                def _():
                    start_pos(ci + NBUF, b)
            return 0

        lax.fori_loop(0, n_chunks // NBUF, outer, 0)
        for b in range(NBUF):
            wait_out(b)

    return body


def kernel(input_ids, token_table, pos_table, gamma, beta):
    b, s = input_ids.shape
    hidden = token_table.shape[1]
    ids = input_ids.reshape(b * s).astype(jnp.int32)
    sc = _build_sc_kernel(b * s, s, hidden)
    out = sc(ids, token_table, pos_table[:s], gamma, beta)
    return out.reshape(b, s, hidden)


# speedup vs baseline: 1.4197x; 1.4197x over previous
import functools

import jax
import jax.numpy as jnp
from jax import lax
from jax.experimental import pallas as pl
from jax.experimental.pallas import tpu as pltpu
from jax.experimental.pallas import tpu_sc as plsc

LANES = 16
CHUNK = 16
NBUF = 2


def _rsqrt_vec(v):
    i = lax.bitcast_convert_type(v, jnp.int32)
    y = lax.bitcast_convert_type(jnp.int32(0x5F3759DF) - (i >> 1), jnp.float32)
    for _ in range(2):
        y = y * (1.5 - 0.5 * v * y * y)
    return y


@functools.lru_cache(maxsize=None)
def _build_sc_kernel(n_rows, seq, hidden):
    info = plsc.get_sparse_core_info()
    nc, ns = info.num_cores, info.num_subcores
    nw = nc * ns
    assert n_rows % (nw * CHUNK) == 0
    rows_per_w = n_rows // nw
    n_chunks = rows_per_w // CHUNK
    assert n_chunks % NBUF == 0 and n_chunks >= 2 * NBUF
    nv = hidden // LANES

    mesh = plsc.VectorSubcoreMesh(core_axis_name="c", subcore_axis_name="s")

    @functools.partial(
        pl.kernel,
        mesh=mesh,
        compiler_params=pltpu.CompilerParams(needs_layout_passes=False),
        out_type=jax.ShapeDtypeStruct((n_rows, hidden), jnp.float32),
        scratch_types=[
            pltpu.VMEM((rows_per_w,), jnp.int32),
            [pltpu.VMEM((CHUNK, hidden), jnp.float32)] * NBUF,
            [pltpu.VMEM((CHUNK, hidden), jnp.float32)] * NBUF,
            [pltpu.VMEM((CHUNK, hidden), jnp.float32)] * NBUF,
            pltpu.VMEM((hidden,), jnp.float32),
            pltpu.VMEM((hidden,), jnp.float32),
            [pltpu.SemaphoreType.DMA] * NBUF,
            [pltpu.SemaphoreType.DMA] * NBUF,
            [pltpu.SemaphoreType.DMA] * NBUF,
        ],
    )
    def body(ids_hbm, tok_hbm, pos_hbm, gamma_hbm, beta_hbm, out_hbm,
             idx_all, xs, ps, os_, g_v, b_v, gsems, psems, osems):
        wid = lax.axis_index("s") * nc + lax.axis_index("c")
        row0 = wid * rows_per_w
        pos0 = lax.rem(row0, seq)

        pltpu.sync_copy(ids_hbm.at[pl.ds(row0, rows_per_w)], idx_all)
        pltpu.sync_copy(gamma_hbm, g_v)
        pltpu.sync_copy(beta_hbm, b_v)

        zero = jnp.zeros((LANES,), jnp.float32)
        inv_h = jnp.float32(1.0 / hidden)

        def start_pos(ci, b):
            pltpu.async_copy(
                pos_hbm.at[pl.ds(pos0 + ci * CHUNK, CHUNK)], ps[b], psems[b])

        def wait_pos(b):
            pltpu.make_async_copy(pos_hbm.at[pl.ds(0, CHUNK)], ps[b],
                                  psems[b]).wait()

        def start_ga(ci, b):
            pltpu.async_copy(
                tok_hbm.at[idx_all.at[pl.ds(ci * CHUNK, CHUNK)]], xs[b],
                gsems[b])

        def wait_ga(b):
            pltpu.make_async_copy(
                tok_hbm.at[idx_all.at[pl.ds(0, CHUNK)]], xs[b],
                gsems[b]).wait()

        def start_out(ci, b):
            pltpu.async_copy(os_[b], out_hbm.at[pl.ds(row0 + ci * CHUNK, CHUNK)],
                             osems[b])

        def wait_out(b):
            pltpu.make_async_copy(os_[b], out_hbm.at[pl.ds(0, CHUNK)],
                                  osems[b]).wait()

        for b in range(NBUF):
            start_pos(b, b)
        start_ga(0, 0)

        dev = [zero] * 2
        for i in range(nv):
            sl = pl.ds(i * LANES, LANES)
            dev[i % 2] = dev[i % 2] + (jnp.abs(g_v[sl] - 1.0) + jnp.abs(b_v[sl]))
        is_identity = jnp.sum(dev[0] + dev[1]) == 0.0

        def compute_chunk(b, fast):
            @plsc.parallel_loop(0, CHUNK, unroll=2)
            def row_body(r):
                s = [zero] * 2
                ss = [zero] * 2
                for i in range(nv):
                    sl = pl.ds(i * LANES, LANES)
                    t = xs[b][r, sl] + ps[b][r, sl]
                    os_[b][r, sl] = t
                    s[i % 2] = s[i % 2] + t
                    ss[i % 2] = ss[i % 2] + t * t
                stot = s[0] + s[1]
                sstot = ss[0] + ss[1]
                mu = jnp.sum(stot) * inv_h
                var = jnp.sum(sstot) * inv_h - mu * mu
                rstd = _rsqrt_vec(zero + (var + jnp.float32(1e-12)))
                c = mu * rstd
                for i in range(nv):
                    sl = pl.ds(i * LANES, LANES)
                    y = os_[b][r, sl] * rstd - c
                    if not fast:
                        y = y * g_v[sl] + b_v[sl]
                    os_[b][r, sl] = y

        def outer(g, _):
            for b in range(NBUF):
                ci = g * NBUF + b
                nb = (b + 1) % NBUF

                @pl.when(ci + 1 < n_chunks)
                def _():
                    start_ga(ci + 1, nb)

                wait_ga(b)
                wait_pos(b)

                @pl.when(ci >= NBUF)
                def _():
                    wait_out(b)

                @pl.when(is_identity)
                def _():
                    compute_chunk(b, True)

                @pl.when(jnp.logical_not(is_identity))
                def _():
                    compute_chunk(b, False)

                start_out(ci, b)

                @pl.when(ci + NBUF < n_chunks)
                def _():
                    start_pos(ci + NBUF, b)
            return 0

        lax.fori_loop(0, n_chunks // NBUF, outer, 0)
        for b in range(NBUF):
            wait_out(b)

    return body


def kernel(input_ids, token_table, pos_table, gamma, beta):
    b, s = input_ids.shape
    hidden = token_table.shape[1]
    ids = input_ids.reshape(b * s).astype(jnp.int32)
    sc = _build_sc_kernel(b * s, s, hidden)
    out = sc(ids, token_table, pos_table[:s], gamma, beta)
    return out.reshape(b, s, hidden)
